# baseline (speedup 1.0000x reference)
import jax
import jax.numpy as jnp
import numpy as np
from jax import lax
from jax.experimental import pallas as pl
from jax.experimental.pallas import tpu as pltpu
from jax.experimental.pallas import tpu_sc as plsc

_B = 16384
_F = 26
_FP = 32
_E = 16
_ROWS_PER_FIELD = 40000
_EPS = 1e-5

_NC, _NS = 2, 16
_NW = _NC * _NS
_BPW = _B // _NW
_PW = _BPW * _FP
_BANDS = _BPW // 8

_G_STREAMS = 16
_G_ROWS = 128
_G_CHUNK = _G_STREAMS * _G_ROWS
_G_GROUPS = _PW // _G_CHUNK


_R = _F * _ROWS_PER_FIELD
_TCOLS = _R // 128
_TG = 8
_TGROUPS = 32


def _sc_transpose_body(tabt_hbm, out_hbm,
                       in0, in1, out0, out1, si0, si1, so0, so1):
    wid = lax.axis_index("s") * _NC + lax.axis_index("c")
    t0 = wid * (_TG * _TGROUPS)
    lanes = lax.iota(jnp.int32, 16)
    slots = ((in0, out0, si0, so0), (in1, out1, si1, so1))

    def tcol(g, j):
        return jnp.minimum(t0 + g * _TG + j, _TCOLS - 1)

    def fire_in(g, in_v, si):
        for j in range(_TG):
            t = tcol(g, j)
            for bd in range(2):
                pltpu.async_copy(
                    tabt_hbm.at[pl.ds(bd * 8, 8), pl.ds(t * 128, 128)],
                    in_v.at[pl.ds(j * 16 + bd * 8, 8), :], si)

    def drain_in(g, in_v, si):
        for j in range(_TG):
            t = tcol(g, j)
            for bd in range(2):
                pltpu.make_async_copy(
                    tabt_hbm.at[pl.ds(bd * 8, 8), pl.ds(t * 128, 128)],
                    in_v.at[pl.ds(j * 16 + bd * 8, 8), :], si).wait()

    def out_copies(g, out_v, so):
        return [(out_v.at[pl.ds(j * 2048, 2048)],
                 out_hbm.at[pl.ds(tcol(g, j) * 2048, 2048)], so)
                for j in range(_TG)]

    fire_in(0, in0, si0)
    fire_in(1, in1, si1)

    @pl.loop(0, _TGROUPS, step=2)
    def _group(i):
        for b in range(2):
            in_v, out_v, si, so = slots[b]
            g = i + b
            drain_in(g, in_v, si)

            @pl.when(g >= 2)
            def _():
                for s, d, sem in out_copies(g - 2, out_v, so):
                    pltpu.make_async_copy(s, d, sem).wait()

            for j in range(_TG):
                row_v = lanes + (j * 16)

                @pl.loop(0, 128, unroll=16)
                def _col(c):
                    c_v = jnp.full((16,), c, jnp.int32)
                    out_v[pl.ds(j * 2048 + c * 16, 16)] = plsc.load_gather(
                        in_v, [row_v, c_v])

            for s, d, sem in out_copies(g, out_v, so):
                pltpu.async_copy(s, d, sem)

            @pl.when(g + 2 < _TGROUPS)
            def _():
                fire_in(g + 2, in_v, si)

    for b in range(2):
        in_v, out_v, si, so = slots[b]
        for s, d, sem in out_copies(_TGROUPS - 2 + b, out_v, so):
            pltpu.make_async_copy(s, d, sem).wait()


def _sc_transpose(tabt):
    mesh = plsc.VectorSubcoreMesh(core_axis_name="c", subcore_axis_name="s")
    call = pl.kernel(
        _sc_transpose_body,
        out_type=jax.ShapeDtypeStruct((_R * _E,), jnp.float32),
        mesh=mesh,
        scratch_types=[
            pltpu.VMEM((_TG * 16, 128), jnp.float32),
            pltpu.VMEM((_TG * 16, 128), jnp.float32),
            pltpu.VMEM((_TG * 2048,), jnp.float32),
            pltpu.VMEM((_TG * 2048,), jnp.float32),
            pltpu.SemaphoreType.DMA,
            pltpu.SemaphoreType.DMA,
            pltpu.SemaphoreType.DMA,
            pltpu.SemaphoreType.DMA,
        ],
        compiler_params=pltpu.CompilerParams(use_tc_tiling_on_sc=True,
                                             needs_layout_passes=False),
    )
    return call(tabt)


def _sc_gather_body(xt_hbm, table_hbm, emb_hbm, x_v, idx_v, rows_v, sem):
    wid = lax.axis_index("s") * _NC + lax.axis_index("c")
    b0 = wid * _BPW
    pltpu.sync_copy(xt_hbm.at[:, pl.ds(b0, _BPW)], x_v)

    lanes = lax.iota(jnp.int32, 16)
    kv = lanes & 7
    rh = lanes >> 3
    f_vs, off_vs = [], []
    for c in range(4):
        fv = kv + (8 * c)
        if c == 3:
            fv = jnp.where(kv < 2, fv, 0)
        f_vs.append(fv)
        off_vs.append(fv * _ROWS_PER_FIELD)
    bb_vs = [rh + (2 * s2) for s2 in range(4)]

    @pl.loop(0, _BANDS)
    def _idx(band):
        base = band * (_FP * 8)
        for c in range(4):
            for s2 in range(4):
                b_v = bb_vs[s2] + band * 8
                xval = plsc.load_gather(x_v, [f_vs[c], b_v])
                idx_v[pl.ds(base + c * 64 + s2 * 16, 16)] = xval + off_vs[c]

    @pl.loop(0, _G_GROUPS)
    def _gather(g):
        gbase = g * _G_CHUNK
        copies = [
            pltpu.async_copy(
                table_hbm.at[idx_v.at[pl.ds(gbase + c * _G_ROWS, _G_ROWS)]],
                rows_v.at[pl.ds(c * _G_ROWS, _G_ROWS)],
                sem,
            )
            for c in range(_G_STREAMS)
        ]
        for cp in copies:
            cp.wait()
        pltpu.sync_copy(rows_v,
                        emb_hbm.at[pl.ds(wid * _PW + gbase, _G_CHUNK)])


def _sc_gather(xt, table):
    mesh = plsc.VectorSubcoreMesh(core_axis_name="c", subcore_axis_name="s")
    call = pl.kernel(
        _sc_gather_body,
        out_type=jax.ShapeDtypeStruct((_B * _FP, _E), jnp.float32),
        mesh=mesh,
        scratch_types=[
            pltpu.VMEM((_F, _BPW), jnp.int32),
            pltpu.VMEM((_PW,), jnp.int32),
            pltpu.VMEM((_G_CHUNK, _E), jnp.float32),
            pltpu.SemaphoreType.DMA,
        ],
        compiler_params=pltpu.CompilerParams(use_tc_tiling_on_sc=False,
                                             needs_layout_passes=False),
    )
    return call(xt, table)


def _mlp_body(emb_ref, w1_ref, b1_ref, w2_ref, b2_ref, w3_ref, b3_ref,
              out_ref):
    v = emb_ref[...]
    bt = v.shape[0] * 8
    h = None
    for c in range(4):
        part = jnp.dot(v[:, c].reshape(bt, 128), w1_ref[c],
                       preferred_element_type=jnp.float32)
        h = part if h is None else h + part
    h = jnp.maximum(h + b1_ref[...], 0.0)
    h = jnp.dot(h, w2_ref[...], preferred_element_type=jnp.float32)
    h = jnp.maximum(h + b2_ref[...], 0.0)
    out_ref[...] = (
        jnp.dot(h, w3_ref[...], preferred_element_type=jnp.float32)
        + b3_ref[...]
    )


_BT = 2048


def _mlp(embv, w1g, b1, w2, b2, w3, b3):
    grid = (_B // _BT,)
    full = lambda *shape: pl.BlockSpec(shape, lambda i: (0,) * len(shape))
    return pl.pallas_call(
        _mlp_body,
        grid=grid,
        in_specs=[
            pl.BlockSpec((_BT // 8, 4, 8, 128), lambda i: (i, 0, 0, 0)),
            full(4, 128, 128),
            full(1, 128),
            full(128, 64),
            full(1, 64),
            full(64, 2),
            full(1, 2),
        ],
        out_specs=pl.BlockSpec((_BT, 2), lambda i: (i, 0)),
        out_shape=jax.ShapeDtypeStruct((_B, 2), jnp.float32),
    )(embv, w1g, b1, w2, b2, w3, b3)


@jax.jit
def kernel(x, table, W1, b1, g1, be1, W2, b2, g2, be2, W3, b3):
    s = np.float32(1.0 / np.sqrt(1.0 + _EPS))
    w1 = (W1 * (g1 * s)[:, None]).T
    w1g = jnp.concatenate([w1, jnp.zeros((96, 128), jnp.float32)],
                          axis=0).reshape(4, 128, 128)
    b1f = ((b1 * g1) * s + be1)[None, :]
    w2 = (W2 * (g2 * s)[:, None]).T
    b2f = ((b2 * g2) * s + be2)[None, :]
    w3 = W3.T
    b3f = b3[None, :]

    tablin = _sc_transpose(table.T).reshape(_R, _E)
    emb = _sc_gather(x.T, tablin)
    embv = emb.reshape(_B // 8, 4, 8, 128)
    return _mlp(embv, w1g, b1f, w2, b2f, w3, b3f)


# speedup vs baseline: 1.0121x; 1.0121x over previous
import jax
import jax.numpy as jnp
import numpy as np
from jax import lax
from jax.experimental import pallas as pl
from jax.experimental.pallas import tpu as pltpu
from jax.experimental.pallas import tpu_sc as plsc

_B = 16384
_F = 26
_FP = 32
_E = 16
_ROWS_PER_FIELD = 40000
_EPS = 1e-5

_NC, _NS = 2, 16
_NW = _NC * _NS
_BPW = _B // _NW
_PW = _BPW * _FP
_BANDS = _BPW // 8

_G_STREAMS = 16
_G_ROWS = 128
_G_CHUNK = _G_STREAMS * _G_ROWS
_G_GROUPS = _PW // _G_CHUNK


_R = _F * _ROWS_PER_FIELD
_TCOLS = _R // 128
_TG = 8
_TGROUPS = 32


def _sc_transpose_body(tabt_hbm, out_hbm,
                       in0, in1, out0, out1, si0, si1, so0, so1):
    wid = lax.axis_index("s") * _NC + lax.axis_index("c")
    t0 = wid * (_TG * _TGROUPS)
    lanes = lax.iota(jnp.int32, 16)
    slots = ((in0, out0, si0, so0), (in1, out1, si1, so1))

    def tcol(g, j):
        return jnp.minimum(t0 + g * _TG + j, _TCOLS - 1)

    def fire_in(g, in_v, si):
        for j in range(_TG):
            t = tcol(g, j)
            for bd in range(2):
                pltpu.async_copy(
                    tabt_hbm.at[pl.ds(bd * 8, 8), pl.ds(t * 128, 128)],
                    in_v.at[pl.ds(j * 16 + bd * 8, 8), :], si)

    def drain_in(g, in_v, si):
        for j in range(_TG):
            t = tcol(g, j)
            for bd in range(2):
                pltpu.make_async_copy(
                    tabt_hbm.at[pl.ds(bd * 8, 8), pl.ds(t * 128, 128)],
                    in_v.at[pl.ds(j * 16 + bd * 8, 8), :], si).wait()

    def out_copies(g, out_v, so):
        return [(out_v.at[pl.ds(j * 2048, 2048)],
                 out_hbm.at[pl.ds(tcol(g, j) * 2048, 2048)], so)
                for j in range(_TG)]

    fire_in(0, in0, si0)
    fire_in(1, in1, si1)

    @pl.loop(0, _TGROUPS, step=2)
    def _group(i):
        for b in range(2):
            in_v, out_v, si, so = slots[b]
            g = i + b
            drain_in(g, in_v, si)

            @pl.when(g >= 2)
            def _():
                for s, d, sem in out_copies(g - 2, out_v, so):
                    pltpu.make_async_copy(s, d, sem).wait()

            @pl.loop(0, _TG)
            def _tile(j):
                row_v = lanes + j * 16
                obase = j * 2048
                for c in range(128):
                    c_v = jnp.full((16,), c, jnp.int32)
                    out_v[pl.ds(obase + c * 16, 16)] = plsc.load_gather(
                        in_v, [row_v, c_v])

            for s, d, sem in out_copies(g, out_v, so):
                pltpu.async_copy(s, d, sem)

            @pl.when(g + 2 < _TGROUPS)
            def _():
                fire_in(g + 2, in_v, si)

    for b in range(2):
        in_v, out_v, si, so = slots[b]
        for s, d, sem in out_copies(_TGROUPS - 2 + b, out_v, so):
            pltpu.make_async_copy(s, d, sem).wait()


def _sc_transpose(tabt):
    mesh = plsc.VectorSubcoreMesh(core_axis_name="c", subcore_axis_name="s")
    call = pl.kernel(
        _sc_transpose_body,
        out_type=jax.ShapeDtypeStruct((_R * _E,), jnp.float32),
        mesh=mesh,
        scratch_types=[
            pltpu.VMEM((_TG * 16, 128), jnp.float32),
            pltpu.VMEM((_TG * 16, 128), jnp.float32),
            pltpu.VMEM((_TG * 2048,), jnp.float32),
            pltpu.VMEM((_TG * 2048,), jnp.float32),
            pltpu.SemaphoreType.DMA,
            pltpu.SemaphoreType.DMA,
            pltpu.SemaphoreType.DMA,
            pltpu.SemaphoreType.DMA,
        ],
        compiler_params=pltpu.CompilerParams(use_tc_tiling_on_sc=True,
                                             needs_layout_passes=False),
    )
    return call(tabt)


def _sc_gather_body(xt_hbm, table_hbm, emb_hbm, x_v, idx_v, rows_v, sem):
    wid = lax.axis_index("s") * _NC + lax.axis_index("c")
    b0 = wid * _BPW
    pltpu.sync_copy(xt_hbm.at[:, pl.ds(b0, _BPW)], x_v)

    lanes = lax.iota(jnp.int32, 16)
    kv = lanes & 7
    rh = lanes >> 3
    f_vs, off_vs = [], []
    for c in range(4):
        fv = kv + (8 * c)
        if c == 3:
            fv = jnp.where(kv < 2, fv, 0)
        f_vs.append(fv)
        off_vs.append(fv * _ROWS_PER_FIELD)
    bb_vs = [rh + (2 * s2) for s2 in range(4)]

    @pl.loop(0, _BANDS)
    def _idx(band):
        base = band * (_FP * 8)
        for c in range(4):
            for s2 in range(4):
                b_v = bb_vs[s2] + band * 8
                xval = plsc.load_gather(x_v, [f_vs[c], b_v])
                idx_v[pl.ds(base + c * 64 + s2 * 16, 16)] = xval + off_vs[c]

    @pl.loop(0, _G_GROUPS)
    def _gather(g):
        gbase = g * _G_CHUNK
        copies = [
            pltpu.async_copy(
                table_hbm.at[idx_v.at[pl.ds(gbase + c * _G_ROWS, _G_ROWS)]],
                rows_v.at[pl.ds(c * _G_ROWS, _G_ROWS)],
                sem,
            )
            for c in range(_G_STREAMS)
        ]
        for cp in copies:
            cp.wait()
        pltpu.sync_copy(rows_v,
                        emb_hbm.at[pl.ds(wid * _PW + gbase, _G_CHUNK)])


def _sc_gather(xt, table):
    mesh = plsc.VectorSubcoreMesh(core_axis_name="c", subcore_axis_name="s")
    call = pl.kernel(
        _sc_gather_body,
        out_type=jax.ShapeDtypeStruct((_B * _FP, _E), jnp.float32),
        mesh=mesh,
        scratch_types=[
            pltpu.VMEM((_F, _BPW), jnp.int32),
            pltpu.VMEM((_PW,), jnp.int32),
            pltpu.VMEM((_G_CHUNK, _E), jnp.float32),
            pltpu.SemaphoreType.DMA,
        ],
        compiler_params=pltpu.CompilerParams(use_tc_tiling_on_sc=False,
                                             needs_layout_passes=False),
    )
    return call(xt, table)


def _mlp_body(emb_ref, w1_ref, b1_ref, w2_ref, b2_ref, w3_ref, b3_ref,
              out_ref):
    v = emb_ref[...]
    bt = v.shape[0] * 8
    h = None
    for c in range(4):
        part = jnp.dot(v[:, c].reshape(bt, 128), w1_ref[c],
                       preferred_element_type=jnp.float32)
        h = part if h is None else h + part
    h = jnp.maximum(h + b1_ref[...], 0.0)
    h = jnp.dot(h, w2_ref[...], preferred_element_type=jnp.float32)
    h = jnp.maximum(h + b2_ref[...], 0.0)
    out_ref[...] = (
        jnp.dot(h, w3_ref[...], preferred_element_type=jnp.float32)
        + b3_ref[...]
    )


_BT = 2048


def _mlp(embv, w1g, b1, w2, b2, w3, b3):
    grid = (_B // _BT,)
    full = lambda *shape: pl.BlockSpec(shape, lambda i: (0,) * len(shape))
    return pl.pallas_call(
        _mlp_body,
        grid=grid,
        in_specs=[
            pl.BlockSpec((_BT // 8, 4, 8, 128), lambda i: (i, 0, 0, 0)),
            full(4, 128, 128),
            full(1, 128),
            full(128, 64),
            full(1, 64),
            full(64, 2),
            full(1, 2),
        ],
        out_specs=pl.BlockSpec((_BT, 2), lambda i: (i, 0)),
        out_shape=jax.ShapeDtypeStruct((_B, 2), jnp.float32),
    )(embv, w1g, b1, w2, b2, w3, b3)


@jax.jit
def kernel(x, table, W1, b1, g1, be1, W2, b2, g2, be2, W3, b3):
    s = np.float32(1.0 / np.sqrt(1.0 + _EPS))
    w1 = (W1 * (g1 * s)[:, None]).T
    w1g = jnp.concatenate([w1, jnp.zeros((96, 128), jnp.float32)],
                          axis=0).reshape(4, 128, 128)
    b1f = ((b1 * g1) * s + be1)[None, :]
    w2 = (W2 * (g2 * s)[:, None]).T
    b2f = ((b2 * g2) * s + be2)[None, :]
    w3 = W3.T
    b3f = b3[None, :]

    tablin = _sc_transpose(table.T).reshape(_R, _E)
    emb = _sc_gather(x.T, tablin)
    embv = emb.reshape(_B // 8, 4, 8, 128)
    return _mlp(embv, w1g, b1f, w2, b2f, w3, b3f)


# speedup vs baseline: 1.6082x; 1.5890x over previous
import jax
import jax.numpy as jnp
import numpy as np
from jax import lax
from jax.experimental import pallas as pl
from jax.experimental.pallas import tpu as pltpu
from jax.experimental.pallas import tpu_sc as plsc

_B = 16384
_F = 26
_FP = 32
_E = 16
_ROWS_PER_FIELD = 40000
_EPS = 1e-5

_NC, _NS = 2, 16
_NW = _NC * _NS
_BPW = _B // _NW
_PW = _BPW * _FP
_BANDS = _BPW // 8

_G_STREAMS = 16
_G_ROWS = 128
_G_CHUNK = _G_STREAMS * _G_ROWS
_G_GROUPS = _PW // _G_CHUNK


_R = _F * _ROWS_PER_FIELD
_TCOLS = _R // 128
_TG = 8
_TGROUPS = 32


def _sc_transpose_body(tabt_hbm, out_hbm,
                       in0, in1, out0, out1, si0, si1, so0, so1):
    wid = lax.axis_index("s") * _NC + lax.axis_index("c")
    t0 = wid * (_TG * _TGROUPS)
    lanes = lax.iota(jnp.int32, 16)
    slots = ((in0, out0, si0, so0), (in1, out1, si1, so1))

    def tcol(g, j):
        return jnp.minimum(t0 + g * _TG + j, _TCOLS - 1)

    def fire_in(g, in_v, si):
        for j in range(_TG):
            t = tcol(g, j)
            for bd in range(2):
                pltpu.async_copy(
                    tabt_hbm.at[pl.ds(bd * 8, 8), pl.ds(t * 128, 128)],
                    in_v.at[pl.ds(j * 16 + bd * 8, 8), :], si)

    def drain_in(g, in_v, si):
        for j in range(_TG):
            t = tcol(g, j)
            for bd in range(2):
                pltpu.make_async_copy(
                    tabt_hbm.at[pl.ds(bd * 8, 8), pl.ds(t * 128, 128)],
                    in_v.at[pl.ds(j * 16 + bd * 8, 8), :], si).wait()

    def out_copies(g, out_v, so):
        return [(out_v.at[pl.ds(j * 2048, 2048)],
                 out_hbm.at[pl.ds(tcol(g, j) * 2048, 2048)], so)
                for j in range(_TG)]

    fire_in(0, in0, si0)
    fire_in(1, in1, si1)

    @pl.loop(0, _TGROUPS, step=2)
    def _group(i):
        for b in range(2):
            in_v, out_v, si, so = slots[b]
            g = i + b
            drain_in(g, in_v, si)

            @pl.when(g >= 2)
            def _():
                for s, d, sem in out_copies(g - 2, out_v, so):
                    pltpu.make_async_copy(s, d, sem).wait()

            @pl.loop(0, _TG)
            def _tile(j):
                row_v = lanes + j * 16
                obase = j * 2048

                @plsc.parallel_loop(0, 128, unroll=8)
                def _col(c):
                    c_v = jnp.full((16,), c, jnp.int32)
                    out_v[pl.ds(obase + c * 16, 16)] = plsc.load_gather(
                        in_v, [row_v, c_v])

            for s, d, sem in out_copies(g, out_v, so):
                pltpu.async_copy(s, d, sem)

            @pl.when(g + 2 < _TGROUPS)
            def _():
                fire_in(g + 2, in_v, si)

    for b in range(2):
        in_v, out_v, si, so = slots[b]
        for s, d, sem in out_copies(_TGROUPS - 2 + b, out_v, so):
            pltpu.make_async_copy(s, d, sem).wait()


def _sc_transpose(tabt):
    mesh = plsc.VectorSubcoreMesh(core_axis_name="c", subcore_axis_name="s")
    call = pl.kernel(
        _sc_transpose_body,
        out_type=jax.ShapeDtypeStruct((_R * _E,), jnp.float32),
        mesh=mesh,
        scratch_types=[
            pltpu.VMEM((_TG * 16, 128), jnp.float32),
            pltpu.VMEM((_TG * 16, 128), jnp.float32),
            pltpu.VMEM((_TG * 2048,), jnp.float32),
            pltpu.VMEM((_TG * 2048,), jnp.float32),
            pltpu.SemaphoreType.DMA,
            pltpu.SemaphoreType.DMA,
            pltpu.SemaphoreType.DMA,
            pltpu.SemaphoreType.DMA,
        ],
        compiler_params=pltpu.CompilerParams(use_tc_tiling_on_sc=True,
                                             needs_layout_passes=False),
    )
    return call(tabt)


def _sc_gather_body(xt_hbm, table_hbm, emb_hbm, x_v, idx_v, rows_v, sem):
    wid = lax.axis_index("s") * _NC + lax.axis_index("c")
    b0 = wid * _BPW
    pltpu.sync_copy(xt_hbm.at[:, pl.ds(b0, _BPW)], x_v)

    lanes = lax.iota(jnp.int32, 16)
    kv = lanes & 7
    rh = lanes >> 3
    f_vs, off_vs = [], []
    for c in range(4):
        fv = kv + (8 * c)
        if c == 3:
            fv = jnp.where(kv < 2, fv, 0)
        f_vs.append(fv)
        off_vs.append(fv * _ROWS_PER_FIELD)
    bb_vs = [rh + (2 * s2) for s2 in range(4)]

    @pl.loop(0, _BANDS)
    def _idx(band):
        base = band * (_FP * 8)
        for c in range(4):
            for s2 in range(4):
                b_v = bb_vs[s2] + band * 8
                xval = plsc.load_gather(x_v, [f_vs[c], b_v])
                idx_v[pl.ds(base + c * 64 + s2 * 16, 16)] = xval + off_vs[c]

    @pl.loop(0, _G_GROUPS)
    def _gather(g):
        gbase = g * _G_CHUNK
        copies = [
            pltpu.async_copy(
                table_hbm.at[idx_v.at[pl.ds(gbase + c * _G_ROWS, _G_ROWS)]],
                rows_v.at[pl.ds(c * _G_ROWS, _G_ROWS)],
                sem,
            )
            for c in range(_G_STREAMS)
        ]
        for cp in copies:
            cp.wait()
        pltpu.sync_copy(rows_v,
                        emb_hbm.at[pl.ds(wid * _PW + gbase, _G_CHUNK)])


def _sc_gather(xt, table):
    mesh = plsc.VectorSubcoreMesh(core_axis_name="c", subcore_axis_name="s")
    call = pl.kernel(
        _sc_gather_body,
        out_type=jax.ShapeDtypeStruct((_B * _FP, _E), jnp.float32),
        mesh=mesh,
        scratch_types=[
            pltpu.VMEM((_F, _BPW), jnp.int32),
            pltpu.VMEM((_PW,), jnp.int32),
            pltpu.VMEM((_G_CHUNK, _E), jnp.float32),
            pltpu.SemaphoreType.DMA,
        ],
        compiler_params=pltpu.CompilerParams(use_tc_tiling_on_sc=False,
                                             needs_layout_passes=False),
    )
    return call(xt, table)


def _mlp_body(emb_ref, w1_ref, b1_ref, w2_ref, b2_ref, w3_ref, b3_ref,
              out_ref):
    v = emb_ref[...]
    bt = v.shape[0] * 8
    h = None
    for c in range(4):
        part = jnp.dot(v[:, c].reshape(bt, 128), w1_ref[c],
                       preferred_element_type=jnp.float32)
        h = part if h is None else h + part
    h = jnp.maximum(h + b1_ref[...], 0.0)
    h = jnp.dot(h, w2_ref[...], preferred_element_type=jnp.float32)
    h = jnp.maximum(h + b2_ref[...], 0.0)
    out_ref[...] = (
        jnp.dot(h, w3_ref[...], preferred_element_type=jnp.float32)
        + b3_ref[...]
    )


_BT = 2048


def _mlp(embv, w1g, b1, w2, b2, w3, b3):
    grid = (_B // _BT,)
    full = lambda *shape: pl.BlockSpec(shape, lambda i: (0,) * len(shape))
    return pl.pallas_call(
        _mlp_body,
        grid=grid,
        in_specs=[
            pl.BlockSpec((_BT // 8, 4, 8, 128), lambda i: (i, 0, 0, 0)),
            full(4, 128, 128),
            full(1, 128),
            full(128, 64),
            full(1, 64),
            full(64, 2),
            full(1, 2),
        ],
        out_specs=pl.BlockSpec((_BT, 2), lambda i: (i, 0)),
        out_shape=jax.ShapeDtypeStruct((_B, 2), jnp.float32),
    )(embv, w1g, b1, w2, b2, w3, b3)


@jax.jit
def kernel(x, table, W1, b1, g1, be1, W2, b2, g2, be2, W3, b3):
    s = np.float32(1.0 / np.sqrt(1.0 + _EPS))
    w1 = (W1 * (g1 * s)[:, None]).T
    w1g = jnp.concatenate([w1, jnp.zeros((96, 128), jnp.float32)],
                          axis=0).reshape(4, 128, 128)
    b1f = ((b1 * g1) * s + be1)[None, :]
    w2 = (W2 * (g2 * s)[:, None]).T
    b2f = ((b2 * g2) * s + be2)[None, :]
    w3 = W3.T
    b3f = b3[None, :]

    tablin = _sc_transpose(table.T).reshape(_R, _E)
    emb = _sc_gather(x.T, tablin)
    embv = emb.reshape(_B // 8, 4, 8, 128)
    return _mlp(embv, w1g, b1f, w2, b2f, w3, b3f)


# speedup vs baseline: 1.6082x; 1.0000x over previous
import jax
import jax.numpy as jnp
import numpy as np
from jax import lax
from jax.experimental import pallas as pl
from jax.experimental.pallas import tpu as pltpu
from jax.experimental.pallas import tpu_sc as plsc

_B = 16384
_F = 26
_FP = 32
_E = 16
_ROWS_PER_FIELD = 40000
_EPS = 1e-5

_NC, _NS = 2, 16
_NW = _NC * _NS
_BPW = _B // _NW
_PW = _BPW * _FP
_BANDS = _BPW // 8

_G_STREAMS = 16
_G_ROWS = 128
_G_CHUNK = _G_STREAMS * _G_ROWS
_G_GROUPS = _PW // _G_CHUNK


_R = _F * _ROWS_PER_FIELD
_TCOLS = _R // 128
_TG = 8
_TGROUPS = 32


def _sc_transpose_body(tabt_hbm, out_hbm,
                       in0, in1, out0, out1, si0, si1, so0, so1):
    wid = lax.axis_index("s") * _NC + lax.axis_index("c")
    t0 = wid * (_TG * _TGROUPS)
    lanes = lax.iota(jnp.int32, 16)
    slots = ((in0, out0, si0, so0), (in1, out1, si1, so1))

    def tcol(g, j):
        return jnp.minimum(t0 + g * _TG + j, _TCOLS - 1)

    def fire_in(g, in_v, si):
        for j in range(_TG):
            t = tcol(g, j)
            for bd in range(2):
                pltpu.async_copy(
                    tabt_hbm.at[pl.ds(bd * 8, 8), pl.ds(t * 128, 128)],
                    in_v.at[pl.ds(j * 16 + bd * 8, 8), :], si)

    def drain_in(g, in_v, si):
        for j in range(_TG):
            t = tcol(g, j)
            for bd in range(2):
                pltpu.make_async_copy(
                    tabt_hbm.at[pl.ds(bd * 8, 8), pl.ds(t * 128, 128)],
                    in_v.at[pl.ds(j * 16 + bd * 8, 8), :], si).wait()

    def out_copies(g, out_v, so):
        return [(out_v.at[pl.ds(j * 2048, 2048)],
                 out_hbm.at[pl.ds(tcol(g, j) * 2048, 2048)], so)
                for j in range(_TG)]

    fire_in(0, in0, si0)
    fire_in(1, in1, si1)

    @pl.loop(0, _TGROUPS, step=2)
    def _group(i):
        for b in range(2):
            in_v, out_v, si, so = slots[b]
            g = i + b
            drain_in(g, in_v, si)

            @pl.when(g >= 2)
            def _():
                for s, d, sem in out_copies(g - 2, out_v, so):
                    pltpu.make_async_copy(s, d, sem).wait()

            @pl.loop(0, _TG)
            def _tile(j):
                row_v = lanes + j * 16
                obase = j * 2048

                @plsc.parallel_loop(0, 128, unroll=16)
                def _col(c):
                    c_v = jnp.full((16,), c, jnp.int32)
                    out_v[pl.ds(obase + c * 16, 16)] = plsc.load_gather(
                        in_v, [row_v, c_v])

            for s, d, sem in out_copies(g, out_v, so):
                pltpu.async_copy(s, d, sem)

            @pl.when(g + 2 < _TGROUPS)
            def _():
                fire_in(g + 2, in_v, si)

    for b in range(2):
        in_v, out_v, si, so = slots[b]
        for s, d, sem in out_copies(_TGROUPS - 2 + b, out_v, so):
            pltpu.make_async_copy(s, d, sem).wait()


def _sc_transpose(tabt):
    mesh = plsc.VectorSubcoreMesh(core_axis_name="c", subcore_axis_name="s")
    call = pl.kernel(
        _sc_transpose_body,
        out_type=jax.ShapeDtypeStruct((_R * _E,), jnp.float32),
        mesh=mesh,
        scratch_types=[
            pltpu.VMEM((_TG * 16, 128), jnp.float32),
            pltpu.VMEM((_TG * 16, 128), jnp.float32),
            pltpu.VMEM((_TG * 2048,), jnp.float32),
            pltpu.VMEM((_TG * 2048,), jnp.float32),
            pltpu.SemaphoreType.DMA,
            pltpu.SemaphoreType.DMA,
            pltpu.SemaphoreType.DMA,
            pltpu.SemaphoreType.DMA,
        ],
        compiler_params=pltpu.CompilerParams(use_tc_tiling_on_sc=True,
                                             needs_layout_passes=False),
    )
    return call(tabt)


def _sc_gather_body(xt_hbm, table_hbm, emb_hbm, x_v, idx_v, rows_v, sem):
    wid = lax.axis_index("s") * _NC + lax.axis_index("c")
    b0 = wid * _BPW
    pltpu.sync_copy(xt_hbm.at[:, pl.ds(b0, _BPW)], x_v)

    lanes = lax.iota(jnp.int32, 16)
    kv = lanes & 7
    rh = lanes >> 3
    f_vs, off_vs = [], []
    for c in range(4):
        fv = kv + (8 * c)
        if c == 3:
            fv = jnp.where(kv < 2, fv, 0)
        f_vs.append(fv)
        off_vs.append(fv * _ROWS_PER_FIELD)
    bb_vs = [rh + (2 * s2) for s2 in range(4)]

    @pl.loop(0, _BANDS)
    def _idx(band):
        base = band * (_FP * 8)
        for c in range(4):
            for s2 in range(4):
                b_v = bb_vs[s2] + band * 8
                xval = plsc.load_gather(x_v, [f_vs[c], b_v])
                idx_v[pl.ds(base + c * 64 + s2 * 16, 16)] = xval + off_vs[c]

    @pl.loop(0, _G_GROUPS)
    def _gather(g):
        gbase = g * _G_CHUNK
        copies = [
            pltpu.async_copy(
                table_hbm.at[idx_v.at[pl.ds(gbase + c * _G_ROWS, _G_ROWS)]],
                rows_v.at[pl.ds(c * _G_ROWS, _G_ROWS)],
                sem,
            )
            for c in range(_G_STREAMS)
        ]
        for cp in copies:
            cp.wait()
        pltpu.sync_copy(rows_v,
                        emb_hbm.at[pl.ds(wid * _PW + gbase, _G_CHUNK)])


def _sc_gather(xt, table):
    mesh = plsc.VectorSubcoreMesh(core_axis_name="c", subcore_axis_name="s")
    call = pl.kernel(
        _sc_gather_body,
        out_type=jax.ShapeDtypeStruct((_B * _FP, _E), jnp.float32),
        mesh=mesh,
        scratch_types=[
            pltpu.VMEM((_F, _BPW), jnp.int32),
            pltpu.VMEM((_PW,), jnp.int32),
            pltpu.VMEM((_G_CHUNK, _E), jnp.float32),
            pltpu.SemaphoreType.DMA,
        ],
        compiler_params=pltpu.CompilerParams(use_tc_tiling_on_sc=False,
                                             needs_layout_passes=False),
    )
    return call(xt, table)


def _mlp_body(emb_ref, w1_ref, b1_ref, w2_ref, b2_ref, w3_ref, b3_ref,
              out_ref):
    v = emb_ref[...]
    bt = v.shape[0] * 8
    h = None
    for c in range(4):
        part = jnp.dot(v[:, c].reshape(bt, 128), w1_ref[c],
                       preferred_element_type=jnp.float32)
        h = part if h is None else h + part
    h = jnp.maximum(h + b1_ref[...], 0.0)
    h = jnp.dot(h, w2_ref[...], preferred_element_type=jnp.float32)
    h = jnp.maximum(h + b2_ref[...], 0.0)
    out_ref[...] = (
        jnp.dot(h, w3_ref[...], preferred_element_type=jnp.float32)
        + b3_ref[...]
    )


_BT = 2048


def _mlp(embv, w1g, b1, w2, b2, w3, b3):
    grid = (_B // _BT,)
    full = lambda *shape: pl.BlockSpec(shape, lambda i: (0,) * len(shape))
    return pl.pallas_call(
        _mlp_body,
        grid=grid,
        in_specs=[
            pl.BlockSpec((_BT // 8, 4, 8, 128), lambda i: (i, 0, 0, 0)),
            full(4, 128, 128),
            full(1, 128),
            full(128, 64),
            full(1, 64),
            full(64, 2),
            full(1, 2),
        ],
        out_specs=pl.BlockSpec((_BT, 2), lambda i: (i, 0)),
        out_shape=jax.ShapeDtypeStruct((_B, 2), jnp.float32),
    )(embv, w1g, b1, w2, b2, w3, b3)


@jax.jit
def kernel(x, table, W1, b1, g1, be1, W2, b2, g2, be2, W3, b3):
    s = np.float32(1.0 / np.sqrt(1.0 + _EPS))
    w1 = (W1 * (g1 * s)[:, None]).T
    w1g = jnp.concatenate([w1, jnp.zeros((96, 128), jnp.float32)],
                          axis=0).reshape(4, 128, 128)
    b1f = ((b1 * g1) * s + be1)[None, :]
    w2 = (W2 * (g2 * s)[:, None]).T
    b2f = ((b2 * g2) * s + be2)[None, :]
    w3 = W3.T
    b3f = b3[None, :]

    tablin = _sc_transpose(table.T).reshape(_R, _E)
    emb = _sc_gather(x.T, tablin)
    embv = emb.reshape(_B // 8, 4, 8, 128)
    return _mlp(embv, w1g, b1f, w2, b2f, w3, b3f)


# speedup vs baseline: 1.6365x; 1.0176x over previous
import jax
import jax.numpy as jnp
import numpy as np
from jax import lax
from jax.experimental import pallas as pl
from jax.experimental.pallas import tpu as pltpu
from jax.experimental.pallas import tpu_sc as plsc

_B = 16384
_F = 26
_FP = 32
_E = 16
_ROWS_PER_FIELD = 40000
_EPS = 1e-5

_NC, _NS = 2, 16
_NW = _NC * _NS
_BPW = _B // _NW
_PW = _BPW * _FP
_BANDS = _BPW // 8

_G_STREAMS = 16
_G_ROWS = 128
_G_CHUNK = _G_STREAMS * _G_ROWS
_G_GROUPS = _PW // _G_CHUNK


_R = _F * _ROWS_PER_FIELD
_TCOLS = _R // 128
_TG = 8
_TGROUPS = 32


def _sc_transpose_body(tabt_hbm, out_hbm,
                       in0, in1, out0, out1, si0, si1, so0, so1):
    wid = lax.axis_index("s") * _NC + lax.axis_index("c")
    t0 = wid * (_TG * _TGROUPS)
    lanes = lax.iota(jnp.int32, 16)
    slots = ((in0, out0, si0, so0), (in1, out1, si1, so1))

    def tcol(g, j):
        return jnp.minimum(t0 + g * _TG + j, _TCOLS - 1)

    def fire_in(g, in_v, si):
        for j in range(_TG):
            t = tcol(g, j)
            for bd in range(2):
                pltpu.async_copy(
                    tabt_hbm.at[pl.ds(bd * 8, 8), pl.ds(t * 128, 128)],
                    in_v.at[pl.ds(j * 16 + bd * 8, 8), pl.ds(0, 128)], si)

    def drain_in(g, in_v, si):
        for j in range(_TG):
            t = tcol(g, j)
            for bd in range(2):
                pltpu.make_async_copy(
                    tabt_hbm.at[pl.ds(bd * 8, 8), pl.ds(t * 128, 128)],
                    in_v.at[pl.ds(j * 16 + bd * 8, 8), pl.ds(0, 128)], si).wait()

    def out_copies(g, out_v, so):
        return [(out_v.at[pl.ds(j * 2048, 2048)],
                 out_hbm.at[pl.ds(tcol(g, j) * 2048, 2048)], so)
                for j in range(_TG)]

    fire_in(0, in0, si0)
    fire_in(1, in1, si1)

    @pl.loop(0, _TGROUPS, step=2)
    def _group(i):
        for b in range(2):
            in_v, out_v, si, so = slots[b]
            g = i + b
            drain_in(g, in_v, si)

            @pl.when(g >= 2)
            def _():
                for s, d, sem in out_copies(g - 2, out_v, so):
                    pltpu.make_async_copy(s, d, sem).wait()

            @pl.loop(0, _TG)
            def _tile(j):
                row_v = lanes + j * 16
                obase = j * 2048

                @plsc.parallel_loop(0, 128, unroll=16)
                def _col(c):
                    c_v = jnp.full((16,), c, jnp.int32)
                    out_v[pl.ds(obase + c * 16, 16)] = plsc.load_gather(
                        in_v, [row_v, c_v])

            for s, d, sem in out_copies(g, out_v, so):
                pltpu.async_copy(s, d, sem)

            @pl.when(g + 2 < _TGROUPS)
            def _():
                fire_in(g + 2, in_v, si)

    for b in range(2):
        in_v, out_v, si, so = slots[b]
        for s, d, sem in out_copies(_TGROUPS - 2 + b, out_v, so):
            pltpu.make_async_copy(s, d, sem).wait()


def _sc_transpose(tabt):
    mesh = plsc.VectorSubcoreMesh(core_axis_name="c", subcore_axis_name="s")
    call = pl.kernel(
        _sc_transpose_body,
        out_type=jax.ShapeDtypeStruct((_R * _E,), jnp.float32),
        mesh=mesh,
        scratch_types=[
            pltpu.VMEM((_TG * 16, 129), jnp.float32),
            pltpu.VMEM((_TG * 16, 129), jnp.float32),
            pltpu.VMEM((_TG * 2048,), jnp.float32),
            pltpu.VMEM((_TG * 2048,), jnp.float32),
            pltpu.SemaphoreType.DMA,
            pltpu.SemaphoreType.DMA,
            pltpu.SemaphoreType.DMA,
            pltpu.SemaphoreType.DMA,
        ],
        compiler_params=pltpu.CompilerParams(use_tc_tiling_on_sc=True,
                                             needs_layout_passes=False),
    )
    return call(tabt)


def _sc_gather_body(xt_hbm, table_hbm, emb_hbm, x_v, idx_v, rows_v, sem):
    wid = lax.axis_index("s") * _NC + lax.axis_index("c")
    b0 = wid * _BPW
    pltpu.sync_copy(xt_hbm.at[:, pl.ds(b0, _BPW)], x_v)

    lanes = lax.iota(jnp.int32, 16)
    kv = lanes & 7
    rh = lanes >> 3
    f_vs, off_vs = [], []
    for c in range(4):
        fv = kv + (8 * c)
        if c == 3:
            fv = jnp.where(kv < 2, fv, 0)
        f_vs.append(fv)
        off_vs.append(fv * _ROWS_PER_FIELD)
    bb_vs = [rh + (2 * s2) for s2 in range(4)]

    @pl.loop(0, _BANDS)
    def _idx(band):
        base = band * (_FP * 8)
        for c in range(4):
            for s2 in range(4):
                b_v = bb_vs[s2] + band * 8
                xval = plsc.load_gather(x_v, [f_vs[c], b_v])
                idx_v[pl.ds(base + c * 64 + s2 * 16, 16)] = xval + off_vs[c]

    @pl.loop(0, _G_GROUPS)
    def _gather(g):
        gbase = g * _G_CHUNK
        copies = [
            pltpu.async_copy(
                table_hbm.at[idx_v.at[pl.ds(gbase + c * _G_ROWS, _G_ROWS)]],
                rows_v.at[pl.ds(c * _G_ROWS, _G_ROWS)],
                sem,
            )
            for c in range(_G_STREAMS)
        ]
        for cp in copies:
            cp.wait()
        pltpu.sync_copy(rows_v,
                        emb_hbm.at[pl.ds(wid * _PW + gbase, _G_CHUNK)])


def _sc_gather(xt, table):
    mesh = plsc.VectorSubcoreMesh(core_axis_name="c", subcore_axis_name="s")
    call = pl.kernel(
        _sc_gather_body,
        out_type=jax.ShapeDtypeStruct((_B * _FP, _E), jnp.float32),
        mesh=mesh,
        scratch_types=[
            pltpu.VMEM((_F, _BPW), jnp.int32),
            pltpu.VMEM((_PW,), jnp.int32),
            pltpu.VMEM((_G_CHUNK, _E), jnp.float32),
            pltpu.SemaphoreType.DMA,
        ],
        compiler_params=pltpu.CompilerParams(use_tc_tiling_on_sc=False,
                                             needs_layout_passes=False),
    )
    return call(xt, table)


def _mlp_body(emb_ref, w1_ref, b1_ref, w2_ref, b2_ref, w3_ref, b3_ref,
              out_ref):
    v = emb_ref[...]
    bt = v.shape[0] * 8
    h = None
    for c in range(4):
        part = jnp.dot(v[:, c].reshape(bt, 128), w1_ref[c],
                       preferred_element_type=jnp.float32)
        h = part if h is None else h + part
    h = jnp.maximum(h + b1_ref[...], 0.0)
    h = jnp.dot(h, w2_ref[...], preferred_element_type=jnp.float32)
    h = jnp.maximum(h + b2_ref[...], 0.0)
    out_ref[...] = (
        jnp.dot(h, w3_ref[...], preferred_element_type=jnp.float32)
        + b3_ref[...]
    )


_BT = 2048


def _mlp(embv, w1g, b1, w2, b2, w3, b3):
    grid = (_B // _BT,)
    full = lambda *shape: pl.BlockSpec(shape, lambda i: (0,) * len(shape))
    return pl.pallas_call(
        _mlp_body,
        grid=grid,
        in_specs=[
            pl.BlockSpec((_BT // 8, 4, 8, 128), lambda i: (i, 0, 0, 0)),
            full(4, 128, 128),
            full(1, 128),
            full(128, 64),
            full(1, 64),
            full(64, 2),
            full(1, 2),
        ],
        out_specs=pl.BlockSpec((_BT, 2), lambda i: (i, 0)),
        out_shape=jax.ShapeDtypeStruct((_B, 2), jnp.float32),
    )(embv, w1g, b1, w2, b2, w3, b3)


@jax.jit
def kernel(x, table, W1, b1, g1, be1, W2, b2, g2, be2, W3, b3):
    s = np.float32(1.0 / np.sqrt(1.0 + _EPS))
    w1 = (W1 * (g1 * s)[:, None]).T
    w1g = jnp.concatenate([w1, jnp.zeros((96, 128), jnp.float32)],
                          axis=0).reshape(4, 128, 128)
    b1f = ((b1 * g1) * s + be1)[None, :]
    w2 = (W2 * (g2 * s)[:, None]).T
    b2f = ((b2 * g2) * s + be2)[None, :]
    w3 = W3.T
    b3f = b3[None, :]

    tablin = _sc_transpose(table.T).reshape(_R, _E)
    emb = _sc_gather(x.T, tablin)
    embv = emb.reshape(_B // 8, 4, 8, 128)
    return _mlp(embv, w1g, b1f, w2, b2f, w3, b3f)
